# Initial kernel scaffold; baseline (speedup 1.0000x reference)
import jax
import jax.numpy as jnp
from jax.experimental import pallas as pl


def kernel(x, W_patch, b_patch, cls_token, pos_embed, ln1_w, ln1_b, W_qkv, b_qkv, W_o, b_o, ln2_w, ln2_b, W_fc1, b_fc1, W_fc2, b_fc2, lnf_w, lnf_b, W_head, b_head):
    raise NotImplementedError("write your pallas kernel here")


# speedup vs baseline: 20.7102x; 20.7102x over previous
import functools

import jax
import jax.numpy as jnp
from jax.experimental import pallas as pl

L = 12
D = 768
H = 12
DH = 64
FF = 3072
P = 16
IMG = 224
NPATCH = (IMG // P) * (IMG // P)
T = NPATCH + 1
NCLS = 1000
K = 128
B = 8

TPAD = 200
ROWS = B * TPAD
EPS = 1e-6
INT_MIN = -2147483648


def _ln_rows(h, w, b):
    m = jnp.mean(h, axis=-1, keepdims=True)
    v = jnp.mean((h - m) * (h - m), axis=-1, keepdims=True)
    return (h - m) * jax.lax.rsqrt(v + EPS) * w + b


def _dot(a, b):
    return jnp.dot(a, b, preferred_element_type=jnp.float32)


def _embed_kernel(p_ref, w_ref, b_ref, o_ref):
    o_ref[...] = _dot(p_ref[...], w_ref[...]) + b_ref[...]


def _attn_kernel(h_ref, lnw_ref, lnb_ref, wqkv_ref, bqkv_ref, wo_ref, bo_ref,
                 o_ref):
    h = h_ref[0]
    hn = _ln_rows(h, lnw_ref[...], lnb_ref[...])
    qkv = _dot(hn, wqkv_ref[...]) + bqkv_ref[...]
    scale = 1.0 / jnp.sqrt(jnp.float32(DH))
    col = jax.lax.broadcasted_iota(jnp.int32, (TPAD, TPAD), 1)
    keymask = col < T
    outs = []
    for hd in range(H):
        q = qkv[:, hd * DH:(hd + 1) * DH]
        k = qkv[:, D + hd * DH:D + (hd + 1) * DH]
        v = qkv[:, 2 * D + hd * DH:2 * D + (hd + 1) * DH]
        s = jax.lax.dot_general(
            q, k, (((1,), (1,)), ((), ())),
            preferred_element_type=jnp.float32) * scale
        s = jnp.where(keymask, s, -1e30)
        smax = jnp.max(s, axis=-1, keepdims=True)
        e = jnp.exp(s - smax)
        pattn = e / jnp.sum(e, axis=-1, keepdims=True)
        outs.append(_dot(pattn, v))
    o = jnp.concatenate(outs, axis=-1)
    o_ref[0] = h + (_dot(o, wo_ref[...]) + bo_ref[...])


def _mlp_kernel(h_ref, lnw_ref, lnb_ref, wfc1_ref, bfc1_ref, wfc2_ref,
                bfc2_ref, o_ref):
    h = h_ref[...]
    hn = _ln_rows(h, lnw_ref[...], lnb_ref[...])
    ffpre = _dot(hn, wfc1_ref[...]) + bfc1_ref[...]
    ff = ffpre * 0.5 * (1.0 + jax.lax.erf(ffpre * (1.0 / jnp.sqrt(2.0))))
    bits = jax.lax.bitcast_convert_type(ff, jnp.int32)
    key = bits ^ ((bits >> 31) & 0x7FFFFFFF)

    def count_ge(c):
        return jnp.sum((key >= c).astype(jnp.int32), axis=-1, keepdims=True)

    zero = jnp.zeros((h.shape[0], 1), jnp.int32)
    t = jnp.where(count_ge(zero) >= K, zero, jnp.full_like(zero, INT_MIN))
    for bit in range(30, -1, -1):
        cand = t | (1 << bit)
        t = jnp.where(count_ge(cand) >= K, cand, t)
    ffm = jnp.where(key >= t, ff, 0.0)
    o_ref[...] = h + (_dot(ffm, wfc2_ref[...]) + bfc2_ref[...])


def _head_kernel(h_ref, lnw_ref, lnb_ref, w_ref, b_ref, o_ref):
    hn = _ln_rows(h_ref[...], lnw_ref[...], lnb_ref[...])
    o_ref[...] = _dot(hn, w_ref[...]) + b_ref[...]


def _full(shape):
    return pl.BlockSpec(shape, lambda *_: tuple(0 for _ in shape))


def kernel(x, W_patch, b_patch, cls_token, pos_embed, ln1_w, ln1_b, W_qkv,
           b_qkv, W_o, b_o, ln2_w, ln2_b, W_fc1, b_fc1, W_fc2, b_fc2, lnf_w,
           lnf_b, W_head, b_head):
    b = x.shape[0]
    g = IMG // P
    p = x.reshape(b, 3, g, P, g, P)
    p = jnp.transpose(p, (0, 2, 4, 1, 3, 5)).reshape(b * NPATCH, 3 * P * P)

    emb = pl.pallas_call(
        _embed_kernel,
        out_shape=jax.ShapeDtypeStruct((b * NPATCH, D), jnp.float32),
        in_specs=[_full((b * NPATCH, 3 * P * P)), _full((3 * P * P, D)),
                  _full((1, D))],
        out_specs=_full((b * NPATCH, D)),
    )(p, W_patch, b_patch.reshape(1, D))

    h = jnp.concatenate(
        [jnp.broadcast_to(cls_token, (b, 1, D)), emb.reshape(b, NPATCH, D)],
        axis=1) + pos_embed
    h = jnp.pad(h, ((0, 0), (0, TPAD - T), (0, 0)))

    attn_call = pl.pallas_call(
        _attn_kernel,
        grid=(B,),
        out_shape=jax.ShapeDtypeStruct((B, TPAD, D), jnp.float32),
        in_specs=[
            pl.BlockSpec((1, TPAD, D), lambda i: (i, 0, 0)),
            _full((1, D)), _full((1, D)),
            _full((D, 3 * D)), _full((1, 3 * D)),
            _full((D, D)), _full((1, D)),
        ],
        out_specs=pl.BlockSpec((1, TPAD, D), lambda i: (i, 0, 0)),
    )

    CHUNK = 200
    mlp_call = pl.pallas_call(
        _mlp_kernel,
        grid=(ROWS // CHUNK,),
        out_shape=jax.ShapeDtypeStruct((ROWS, D), jnp.float32),
        in_specs=[
            pl.BlockSpec((CHUNK, D), lambda i: (i, 0)),
            _full((1, D)), _full((1, D)),
            _full((D, FF)), _full((1, FF)),
            _full((FF, D)), _full((1, D)),
        ],
        out_specs=pl.BlockSpec((CHUNK, D), lambda i: (i, 0)),
    )

    for i in range(L):
        h = attn_call(h, ln1_w[i].reshape(1, D), ln1_b[i].reshape(1, D),
                      W_qkv[i], b_qkv[i].reshape(1, 3 * D), W_o[i],
                      b_o[i].reshape(1, D))
        h2 = mlp_call(h.reshape(ROWS, D), ln2_w[i].reshape(1, D),
                      ln2_b[i].reshape(1, D), W_fc1[i],
                      b_fc1[i].reshape(1, FF), W_fc2[i],
                      b_fc2[i].reshape(1, D))
        h = h2.reshape(B, TPAD, D)

    cls = h[:, 0, :]
    logits = pl.pallas_call(
        _head_kernel,
        out_shape=jax.ShapeDtypeStruct((B, NCLS), jnp.float32),
        in_specs=[_full((B, D)), _full((1, D)), _full((1, D)),
                  _full((D, NCLS)), _full((1, NCLS))],
        out_specs=_full((B, NCLS)),
    )(cls, lnf_w.reshape(1, D), lnf_b.reshape(1, D), W_head,
      b_head.reshape(1, NCLS))
    return logits
